# baseline (speedup 1.0000x reference)
import functools

import jax
import jax.numpy as jnp
from jax import lax
from jax.experimental import pallas as pl
from jax.experimental.pallas import tpu as pltpu
from jax.experimental.pallas import tpu_sc as plsc

T = 2048
D = 1024
F = 256
E = 64
K = 2
FS = 2048
TM = 64
NT = (T * K) // TM + E
NP = NT * TM
DP = D // 2

NC, NS = 2, 16
NW = NC * NS


def _cumsum_rows(a):
    n = a.shape[0]
    s = 1
    while s < n:
        a = a + jnp.concatenate(
            [jnp.zeros((s, a.shape[1]), a.dtype), a[:-s]], axis=0)
        s *= 2
    return a


def _route_body(x_ref, gw_ref, pos_ref, w_ref, texp_ref):
    xf = x_ref[...]
    gw = gw_ref[...]
    logits = lax.dot_general(xf, gw, (((1,), (1,)), ((), ())),
                             preferred_element_type=jnp.float32)
    m = jnp.max(logits, axis=-1, keepdims=True)
    ex = jnp.exp(logits - m)
    sc = ex / jnp.sum(ex, axis=-1, keepdims=True)

    lane = lax.broadcasted_iota(jnp.int32, (T, E), 1)
    m1 = jnp.max(sc, axis=-1, keepdims=True)
    i1 = jnp.min(jnp.where(sc == m1, lane, E), axis=-1, keepdims=True)
    sc2 = jnp.where(lane == i1, -1.0, sc)
    m2 = jnp.max(sc2, axis=-1, keepdims=True)
    i2 = jnp.min(jnp.where(sc2 == m2, lane, E), axis=-1, keepdims=True)

    d = jnp.exp(m2 - m1)
    w1 = 1.0 / (1.0 + d)
    w2 = d / (1.0 + d)

    oh1 = (lane == i1).astype(jnp.float32)
    oh2 = (lane == i2).astype(jnp.float32)
    cum1 = _cumsum_rows(oh1)
    cum2 = _cumsum_rows(oh2)
    c1 = cum1[T - 1:T, :]
    counts = c1 + cum2[T - 1:T, :]

    rank1 = jnp.sum(oh1 * cum1, axis=-1, keepdims=True) - 1.0
    rank2 = jnp.sum(oh2 * (cum2 + c1), axis=-1, keepdims=True) - 1.0

    nt = (counts.astype(jnp.int32) + (TM - 1)) // TM
    a64 = lax.broadcasted_iota(jnp.int32, (E, E), 0)
    b64 = lax.broadcasted_iota(jnp.int32, (E, E), 1)
    tri = (a64 < b64).astype(jnp.float32)
    po_f = lax.dot_general(nt.astype(jnp.float32), tri,
                           (((1,), (0,)), ((), ())),
                           preferred_element_type=jnp.float32) * TM

    pos1 = (jnp.sum(oh1 * po_f, axis=-1, keepdims=True)
            + rank1).astype(jnp.int32)
    pos2 = (jnp.sum(oh2 * po_f, axis=-1, keepdims=True)
            + rank2).astype(jnp.int32)

    pos_ref[...] = jnp.concatenate([pos1, pos2], axis=1)
    w_ref[...] = jnp.concatenate([w1, w2], axis=1)

    jt = lax.broadcasted_iota(jnp.int32, (NT, E), 0)
    start = po_f.astype(jnp.int32) // TM
    ind = (jt >= start) & (jt < start + nt)
    lane2 = lax.broadcasted_iota(jnp.int32, (NT, E), 1)
    texp_ref[...] = jnp.sum(jnp.where(ind, lane2, 0), axis=1,
                            keepdims=True)


def _route(xf, gate_w):
    return pl.pallas_call(
        _route_body,
        out_shape=[
            jax.ShapeDtypeStruct((T, K), jnp.int32),
            jax.ShapeDtypeStruct((T, K), jnp.float32),
            jax.ShapeDtypeStruct((NT, 1), jnp.int32),
        ],
    )(xf, gate_w)


def _scatter_body(pos_hbm, wp_hbm, src_hbm, wrow_hbm,
                  posv, wpv, srcv, wrv, sem):
    wid = lax.axis_index("s") * NC + lax.axis_index("c")

    @pl.when(wid == 0)
    def _():
        pltpu.sync_copy(pos_hbm, posv)
        pltpu.sync_copy(wp_hbm, wpv)

        def init(i, _):
            srcv[pl.ds(i * 16, 16)] = jnp.zeros((16,), jnp.int32)
            wrv[pl.ds(i * 16, 16)] = jnp.zeros((16,), jnp.float32)
            return 0
        lax.fori_loop(0, NP // 16, init, 0)

        iota = lax.iota(jnp.int32, 16)

        def scat(i, _):
            off = i * 16
            idx = posv[pl.ds(off, 16)]
            tok = (off + iota) >> 1
            wv16 = wpv[pl.ds(off, 16)]
            plsc.store_scatter(srcv, [idx], tok)
            plsc.store_scatter(wrv, [idx], wv16)
            return 0
        lax.fori_loop(0, (T * K) // 16, scat, 0)

        pltpu.sync_copy(srcv, src_hbm)
        pltpu.sync_copy(wrv, wrow_hbm)


def _scatter(posf, wf):
    mesh = plsc.VectorSubcoreMesh(core_axis_name="c", subcore_axis_name="s")
    return pl.kernel(
        _scatter_body,
        out_type=[
            jax.ShapeDtypeStruct((NP,), jnp.int32),
            jax.ShapeDtypeStruct((NP,), jnp.float32),
        ],
        mesh=mesh,
        compiler_params=pltpu.CompilerParams(needs_layout_passes=False),
        scratch_types=[
            pltpu.VMEM((T * K,), jnp.int32),
            pltpu.VMEM((T * K,), jnp.float32),
            pltpu.VMEM((NP,), jnp.int32),
            pltpu.VMEM((NP,), jnp.float32),
            pltpu.SemaphoreType.DMA,
        ],
    )(posf, wf)


_GCH = 32
_GPW = NP // NW
_GNC = _GPW // _GCH
_GNB = 4


def _gather_body(src_hbm, x_hbm, xs_hbm, idxv, r0, r1, r2, r3,
                 s0, s1, s2, s3, osem):
    wid = lax.axis_index("s") * NC + lax.axis_index("c")
    base = wid * _GPW
    pltpu.sync_copy(src_hbm.at[pl.ds(base, _GPW)], idxv)
    bufs = (r0, r1, r2, r3)
    sems = (s0, s1, s2, s3)

    def fire(g):
        return pltpu.async_copy(
            x_hbm.at[idxv.at[pl.ds(g * _GCH, _GCH)]], bufs[g % _GNB],
            sems[g % _GNB])

    copies = [fire(g) for g in range(_GNB)]
    outs = []
    for g in range(_GNC):
        copies[g].wait()
        outs.append(pltpu.async_copy(
            bufs[g % _GNB], xs_hbm.at[pl.ds(base + g * _GCH, _GCH)], osem))
        if g + _GNB < _GNC:
            outs[g].wait()
            copies.append(fire(g + _GNB))
    for g in range(max(0, _GNC - _GNB), _GNC):
        outs[g].wait()


def _gather(src, xf):
    mesh = plsc.VectorSubcoreMesh(core_axis_name="c", subcore_axis_name="s")
    return pl.kernel(
        _gather_body,
        out_type=jax.ShapeDtypeStruct((NP, DP), jnp.uint32),
        mesh=mesh,
        compiler_params=pltpu.CompilerParams(needs_layout_passes=False),
        scratch_types=[
            pltpu.VMEM((_GPW,), jnp.int32),
            pltpu.VMEM((_GCH, DP), jnp.uint32),
            pltpu.VMEM((_GCH, DP), jnp.uint32),
            pltpu.VMEM((_GCH, DP), jnp.uint32),
            pltpu.VMEM((_GCH, DP), jnp.uint32),
            pltpu.SemaphoreType.DMA,
            pltpu.SemaphoreType.DMA,
            pltpu.SemaphoreType.DMA,
            pltpu.SemaphoreType.DMA,
            pltpu.SemaphoreType.DMA,
        ],
    )(src, xf)


def _ffn_body(texp_ref, xs_ref, w_ref, w1_ref, b1_ref, w2_ref, b2_ref,
              w3_ref, b3_ref, ys_ref):
    xt = xs_ref[...]
    w1 = w1_ref[0].astype(jnp.bfloat16)
    w3 = w3_ref[0].astype(jnp.bfloat16)
    w2 = w2_ref[0].astype(jnp.bfloat16)
    h1 = lax.dot_general(xt, w1, (((1,), (1,)), ((), ())),
                         preferred_element_type=jnp.float32) + b1_ref[0]
    h3 = lax.dot_general(xt, w3, (((1,), (1,)), ((), ())),
                         preferred_element_type=jnp.float32) + b3_ref[0]
    p = h1 * h3
    h = (p / (1.0 + jnp.exp(-p))).astype(jnp.bfloat16)
    out = lax.dot_general(h, w2, (((1,), (1,)), ((), ())),
                          preferred_element_type=jnp.float32) + b2_ref[0]
    ys_ref[...] = out * w_ref[...]


def _grouped_ffn(texp, xs, wrow, W1, b1, W2, b2, W3, b3):
    grid_spec = pltpu.PrefetchScalarGridSpec(
        num_scalar_prefetch=1,
        grid=(NT,),
        in_specs=[
            pl.BlockSpec((TM, D), lambda j, te: (j, 0)),
            pl.BlockSpec((TM, 1), lambda j, te: (j, 0)),
            pl.BlockSpec((1, F, D), lambda j, te: (te[j], 0, 0)),
            pl.BlockSpec((1, 1, F), lambda j, te: (te[j], 0, 0)),
            pl.BlockSpec((1, D, F), lambda j, te: (te[j], 0, 0)),
            pl.BlockSpec((1, 1, D), lambda j, te: (te[j], 0, 0)),
            pl.BlockSpec((1, F, D), lambda j, te: (te[j], 0, 0)),
            pl.BlockSpec((1, 1, F), lambda j, te: (te[j], 0, 0)),
        ],
        out_specs=pl.BlockSpec((TM, D), lambda j, te: (j, 0)),
    )
    return pl.pallas_call(
        _ffn_body,
        grid_spec=grid_spec,
        out_shape=jax.ShapeDtypeStruct((NP, D), jnp.float32),
    )(texp, xs, wrow, W1, b1, W2, b2, W3, b3)


def _shared_body(x_ref, w1_ref, b1_ref, w2_ref, b2_ref, w3_ref, b3_ref,
                 s_ref):
    xt = x_ref[...].astype(jnp.bfloat16)
    h1 = lax.dot_general(xt, w1_ref[...], (((1,), (1,)), ((), ())),
                         preferred_element_type=jnp.float32) + b1_ref[...]
    h3 = lax.dot_general(xt, w3_ref[...], (((1,), (1,)), ((), ())),
                         preferred_element_type=jnp.float32) + b3_ref[...]
    p = h1 * h3
    h = (p / (1.0 + jnp.exp(-p))).astype(jnp.bfloat16)
    s_ref[...] = lax.dot_general(h, w2_ref[...], (((1,), (1,)), ((), ())),
                                 preferred_element_type=jnp.float32) \
        + b2_ref[...]


_ET = 128


def _shared_ffn(xf, Ws1, bs1, Ws2, bs2, Ws3, bs3):
    return pl.pallas_call(
        _shared_body,
        grid=(T // _ET,),
        in_specs=[
            pl.BlockSpec((_ET, D), lambda i: (i, 0)),
            pl.BlockSpec((FS, D), lambda i: (0, 0)),
            pl.BlockSpec((FS,), lambda i: (0,)),
            pl.BlockSpec((D, FS), lambda i: (0, 0)),
            pl.BlockSpec((D,), lambda i: (0,)),
            pl.BlockSpec((FS, D), lambda i: (0, 0)),
            pl.BlockSpec((FS,), lambda i: (0,)),
        ],
        out_specs=pl.BlockSpec((_ET, D), lambda i: (i, 0)),
        out_shape=jax.ShapeDtypeStruct((T, D), jnp.float32),
    )(xf, Ws1, bs1, Ws2, bs2, Ws3, bs3)


_CTOK = 8
_TPW = T // NW


_FNC = _TPW // _CTOK


def _combine_body(pos_hbm, ys_hbm, s_hbm, y_hbm,
                  idxv, g0, g1, s0, s1, gs0, gs1, ss0, ss1):
    wid = lax.axis_index("s") * NC + lax.axis_index("c")
    tb0 = wid * _TPW
    pltpu.sync_copy(pos_hbm.at[pl.ds(tb0 * K, _TPW * K)], idxv)
    gb, sb = (g0, g1), (s0, s1)
    gsem, ssem = (gs0, gs1), (ss0, ss1)
    gcop, scop = [], []

    def combine_chunk(c):
        gcop[c].wait()
        scop[c].wait()
        gv, sv = gb[c % 2], sb[c % 2]
        for r in range(_CTOK):
            def body(c2, _, r=r, gv=gv, sv=sv):
                off = c2 * 16
                yv = (gv[2 * r, pl.ds(off, 16)]
                      + gv[2 * r + 1, pl.ds(off, 16)]
                      + sv[r, pl.ds(off, 16)])
                sv[r, pl.ds(off, 16)] = yv
                return 0
            lax.fori_loop(0, D // 16, body, 0)
        pltpu.sync_copy(sv, y_hbm.at[pl.ds(tb0 + c * _CTOK, _CTOK)])

    for ch in range(_FNC):
        gcop.append(pltpu.async_copy(
            ys_hbm.at[idxv.at[pl.ds(ch * _CTOK * K, _CTOK * K)]],
            gb[ch % 2], gsem[ch % 2]))
        scop.append(pltpu.async_copy(
            s_hbm.at[pl.ds(tb0 + ch * _CTOK, _CTOK)],
            sb[ch % 2], ssem[ch % 2]))
        if ch > 0:
            combine_chunk(ch - 1)
    combine_chunk(_FNC - 1)


def _combine(posf, ys, s):
    mesh = plsc.VectorSubcoreMesh(core_axis_name="c", subcore_axis_name="s")
    return pl.kernel(
        _combine_body,
        out_type=jax.ShapeDtypeStruct((T, D), jnp.float32),
        mesh=mesh,
        compiler_params=pltpu.CompilerParams(needs_layout_passes=False),
        scratch_types=[
            pltpu.VMEM((_TPW * K,), jnp.int32),
            pltpu.VMEM((_CTOK * K, D), jnp.float32),
            pltpu.VMEM((_CTOK * K, D), jnp.float32),
            pltpu.VMEM((_CTOK, D), jnp.float32),
            pltpu.VMEM((_CTOK, D), jnp.float32),
            pltpu.SemaphoreType.DMA,
            pltpu.SemaphoreType.DMA,
            pltpu.SemaphoreType.DMA,
            pltpu.SemaphoreType.DMA,
        ],
    )(posf, ys, s)


def kernel(x, gate_w, W1, b1, W2, b2, W3, b3, Ws1, bs1, Ws2, bs2, Ws3, bs3):
    shape = x.shape
    xf = x.reshape(-1, shape[-1])

    pos, wpair, texp2 = _route(xf, gate_w)
    texp = texp2.reshape(NT)
    posf = pos.reshape(T * K)
    wf = wpair.reshape(T * K)

    xpack = lax.bitcast_convert_type(
        xf.astype(jnp.bfloat16).reshape(T, DP, 2), jnp.uint32)
    src, wrow = _scatter(posf, wf)
    xs_pack = _gather(src, xpack)
    xsb = lax.bitcast_convert_type(xs_pack, jnp.bfloat16).reshape(NP, D)
    ys = _grouped_ffn(texp, xsb, wrow.reshape(NP, 1),
                      W1, b1.reshape(E, 1, F), W2, b2.reshape(E, 1, D),
                      W3, b3.reshape(E, 1, F))
    s = _shared_ffn(xf, Ws1.astype(jnp.bfloat16), bs1,
                    Ws2.astype(jnp.bfloat16), bs2,
                    Ws3.astype(jnp.bfloat16), bs3)
    y = _combine(posf, ys, s)
    return y.reshape(shape)


# speedup vs baseline: 1.0020x; 1.0020x over previous
import functools

import jax
import jax.numpy as jnp
from jax import lax
from jax.experimental import pallas as pl
from jax.experimental.pallas import tpu as pltpu
from jax.experimental.pallas import tpu_sc as plsc

T = 2048
D = 1024
F = 256
E = 64
K = 2
FS = 2048
TM = 64
NT = (T * K) // TM + E
NP = NT * TM
DP = D // 2

NC, NS = 2, 16
NW = NC * NS


def _cumsum_rows(a):
    n = a.shape[0]
    s = 1
    while s < n:
        a = a + jnp.concatenate(
            [jnp.zeros((s, a.shape[1]), a.dtype), a[:-s]], axis=0)
        s *= 2
    return a


def _route_body(x_ref, gw_ref, pos_ref, w_ref, texp_ref):
    xf = x_ref[...]
    gw = gw_ref[...]
    logits = lax.dot_general(xf, gw, (((1,), (1,)), ((), ())),
                             preferred_element_type=jnp.float32)
    m = jnp.max(logits, axis=-1, keepdims=True)
    ex = jnp.exp(logits - m)
    sc = ex / jnp.sum(ex, axis=-1, keepdims=True)

    lane = lax.broadcasted_iota(jnp.int32, (T, E), 1)
    m1 = jnp.max(sc, axis=-1, keepdims=True)
    i1 = jnp.min(jnp.where(sc == m1, lane, E), axis=-1, keepdims=True)
    sc2 = jnp.where(lane == i1, -1.0, sc)
    m2 = jnp.max(sc2, axis=-1, keepdims=True)
    i2 = jnp.min(jnp.where(sc2 == m2, lane, E), axis=-1, keepdims=True)

    d = jnp.exp(m2 - m1)
    w1 = 1.0 / (1.0 + d)
    w2 = d / (1.0 + d)

    oh1 = (lane == i1).astype(jnp.float32)
    oh2 = (lane == i2).astype(jnp.float32)
    cum1 = _cumsum_rows(oh1)
    cum2 = _cumsum_rows(oh2)
    c1 = cum1[T - 1:T, :]
    counts = c1 + cum2[T - 1:T, :]

    rank1 = jnp.sum(oh1 * cum1, axis=-1, keepdims=True) - 1.0
    rank2 = jnp.sum(oh2 * (cum2 + c1), axis=-1, keepdims=True) - 1.0

    nt = (counts.astype(jnp.int32) + (TM - 1)) // TM
    a64 = lax.broadcasted_iota(jnp.int32, (E, E), 0)
    b64 = lax.broadcasted_iota(jnp.int32, (E, E), 1)
    tri = (a64 < b64).astype(jnp.float32)
    po_f = lax.dot_general(nt.astype(jnp.float32), tri,
                           (((1,), (0,)), ((), ())),
                           preferred_element_type=jnp.float32) * TM

    pos1 = (jnp.sum(oh1 * po_f, axis=-1, keepdims=True)
            + rank1).astype(jnp.int32)
    pos2 = (jnp.sum(oh2 * po_f, axis=-1, keepdims=True)
            + rank2).astype(jnp.int32)

    pos_ref[...] = jnp.concatenate([pos1, pos2], axis=1)
    w_ref[...] = jnp.concatenate([w1, w2], axis=1)

    jt = lax.broadcasted_iota(jnp.int32, (NT, E), 0)
    start = po_f.astype(jnp.int32) // TM
    ind = (jt >= start) & (jt < start + nt)
    lane2 = lax.broadcasted_iota(jnp.int32, (NT, E), 1)
    texp_ref[...] = jnp.sum(jnp.where(ind, lane2, 0), axis=1,
                            keepdims=True)


def _route(xf, gate_w):
    return pl.pallas_call(
        _route_body,
        out_shape=[
            jax.ShapeDtypeStruct((T, K), jnp.int32),
            jax.ShapeDtypeStruct((T, K), jnp.float32),
            jax.ShapeDtypeStruct((NT, 1), jnp.int32),
        ],
    )(xf, gate_w)


def _scatter_body(pos_hbm, wp_hbm, src_hbm, wrow_hbm,
                  posv, wpv, srcv, wrv, sem):
    wid = lax.axis_index("s") * NC + lax.axis_index("c")

    @pl.when(wid == 0)
    def _():
        pltpu.sync_copy(pos_hbm, posv)
        pltpu.sync_copy(wp_hbm, wpv)

        def init(i, _):
            srcv[pl.ds(i * 16, 16)] = jnp.zeros((16,), jnp.int32)
            wrv[pl.ds(i * 16, 16)] = jnp.zeros((16,), jnp.float32)
            return 0
        lax.fori_loop(0, NP // 16, init, 0)

        iota = lax.iota(jnp.int32, 16)

        def scat(i, _):
            off = i * 16
            idx = posv[pl.ds(off, 16)]
            tok = (off + iota) >> 1
            wv16 = wpv[pl.ds(off, 16)]
            plsc.store_scatter(srcv, [idx], tok)
            plsc.store_scatter(wrv, [idx], wv16)
            return 0
        lax.fori_loop(0, (T * K) // 16, scat, 0)

        pltpu.sync_copy(srcv, src_hbm)
        pltpu.sync_copy(wrv, wrow_hbm)


def _scatter(posf, wf):
    mesh = plsc.VectorSubcoreMesh(core_axis_name="c", subcore_axis_name="s")
    return pl.kernel(
        _scatter_body,
        out_type=[
            jax.ShapeDtypeStruct((NP,), jnp.int32),
            jax.ShapeDtypeStruct((NP,), jnp.float32),
        ],
        mesh=mesh,
        compiler_params=pltpu.CompilerParams(needs_layout_passes=False),
        scratch_types=[
            pltpu.VMEM((T * K,), jnp.int32),
            pltpu.VMEM((T * K,), jnp.float32),
            pltpu.VMEM((NP,), jnp.int32),
            pltpu.VMEM((NP,), jnp.float32),
            pltpu.SemaphoreType.DMA,
        ],
    )(posf, wf)


_GCH = 32
_GPW = NP // NW
_GNC = _GPW // _GCH
_GNB = 4


def _gather_body(src_hbm, x_hbm, xs_hbm, idxv, r0, r1, r2, r3,
                 s0, s1, s2, s3, osem):
    wid = lax.axis_index("s") * NC + lax.axis_index("c")
    base = wid * _GPW
    pltpu.sync_copy(src_hbm.at[pl.ds(base, _GPW)], idxv)
    bufs = (r0, r1, r2, r3)
    sems = (s0, s1, s2, s3)

    def fire(g):
        return pltpu.async_copy(
            x_hbm.at[idxv.at[pl.ds(g * _GCH, _GCH)]], bufs[g % _GNB],
            sems[g % _GNB])

    copies = [fire(g) for g in range(_GNB)]
    outs = []
    for g in range(_GNC):
        copies[g].wait()
        outs.append(pltpu.async_copy(
            bufs[g % _GNB], xs_hbm.at[pl.ds(base + g * _GCH, _GCH)], osem))
        if g + _GNB < _GNC:
            outs[g].wait()
            copies.append(fire(g + _GNB))
    for g in range(max(0, _GNC - _GNB), _GNC):
        outs[g].wait()


def _gather(src, xf):
    mesh = plsc.VectorSubcoreMesh(core_axis_name="c", subcore_axis_name="s")
    return pl.kernel(
        _gather_body,
        out_type=jax.ShapeDtypeStruct((NP, DP), jnp.uint32),
        mesh=mesh,
        compiler_params=pltpu.CompilerParams(needs_layout_passes=False),
        scratch_types=[
            pltpu.VMEM((_GPW,), jnp.int32),
            pltpu.VMEM((_GCH, DP), jnp.uint32),
            pltpu.VMEM((_GCH, DP), jnp.uint32),
            pltpu.VMEM((_GCH, DP), jnp.uint32),
            pltpu.VMEM((_GCH, DP), jnp.uint32),
            pltpu.SemaphoreType.DMA,
            pltpu.SemaphoreType.DMA,
            pltpu.SemaphoreType.DMA,
            pltpu.SemaphoreType.DMA,
            pltpu.SemaphoreType.DMA,
        ],
    )(src, xf)


def _ffn_body(texp_ref, xs_ref, w_ref, w1_ref, b1_ref, w2_ref, b2_ref,
              w3_ref, b3_ref, ys_ref):
    xt = xs_ref[...].astype(jnp.float32)
    h1 = lax.dot_general(xt, w1_ref[0], (((1,), (1,)), ((), ())),
                         preferred_element_type=jnp.float32) + b1_ref[0]
    h3 = lax.dot_general(xt, w3_ref[0], (((1,), (1,)), ((), ())),
                         preferred_element_type=jnp.float32) + b3_ref[0]
    p = h1 * h3
    h = p / (1.0 + jnp.exp(-p))
    out = lax.dot_general(h, w2_ref[0], (((1,), (1,)), ((), ())),
                          preferred_element_type=jnp.float32) + b2_ref[0]
    ys_ref[...] = out * w_ref[...]


def _grouped_ffn(texp, xs, wrow, W1, b1, W2, b2, W3, b3):
    grid_spec = pltpu.PrefetchScalarGridSpec(
        num_scalar_prefetch=1,
        grid=(NT,),
        in_specs=[
            pl.BlockSpec((TM, D), lambda j, te: (j, 0)),
            pl.BlockSpec((TM, 1), lambda j, te: (j, 0)),
            pl.BlockSpec((1, F, D), lambda j, te: (te[j], 0, 0)),
            pl.BlockSpec((1, 1, F), lambda j, te: (te[j], 0, 0)),
            pl.BlockSpec((1, D, F), lambda j, te: (te[j], 0, 0)),
            pl.BlockSpec((1, 1, D), lambda j, te: (te[j], 0, 0)),
            pl.BlockSpec((1, F, D), lambda j, te: (te[j], 0, 0)),
            pl.BlockSpec((1, 1, F), lambda j, te: (te[j], 0, 0)),
        ],
        out_specs=pl.BlockSpec((TM, D), lambda j, te: (j, 0)),
    )
    return pl.pallas_call(
        _ffn_body,
        grid_spec=grid_spec,
        out_shape=jax.ShapeDtypeStruct((NP, D), jnp.float32),
    )(texp, xs, wrow, W1, b1, W2, b2, W3, b3)


def _shared_body(x_ref, w1_ref, b1_ref, w2_ref, b2_ref, w3_ref, b3_ref,
                 s_ref):
    xt = x_ref[...].astype(jnp.bfloat16)
    h1 = lax.dot_general(xt, w1_ref[...], (((1,), (1,)), ((), ())),
                         preferred_element_type=jnp.float32) + b1_ref[...]
    h3 = lax.dot_general(xt, w3_ref[...], (((1,), (1,)), ((), ())),
                         preferred_element_type=jnp.float32) + b3_ref[...]
    p = h1 * h3
    h = (p / (1.0 + jnp.exp(-p))).astype(jnp.bfloat16)
    s_ref[...] = lax.dot_general(h, w2_ref[...], (((1,), (1,)), ((), ())),
                                 preferred_element_type=jnp.float32) \
        + b2_ref[...]


_ET = 128


def _shared_ffn(xf, Ws1, bs1, Ws2, bs2, Ws3, bs3):
    return pl.pallas_call(
        _shared_body,
        grid=(T // _ET,),
        in_specs=[
            pl.BlockSpec((_ET, D), lambda i: (i, 0)),
            pl.BlockSpec((FS, D), lambda i: (0, 0)),
            pl.BlockSpec((FS,), lambda i: (0,)),
            pl.BlockSpec((D, FS), lambda i: (0, 0)),
            pl.BlockSpec((D,), lambda i: (0,)),
            pl.BlockSpec((FS, D), lambda i: (0, 0)),
            pl.BlockSpec((FS,), lambda i: (0,)),
        ],
        out_specs=pl.BlockSpec((_ET, D), lambda i: (i, 0)),
        out_shape=jax.ShapeDtypeStruct((T, D), jnp.float32),
    )(xf, Ws1, bs1, Ws2, bs2, Ws3, bs3)


_CTOK = 8
_TPW = T // NW


_FNC = _TPW // _CTOK


def _combine_body(pos_hbm, ys_hbm, s_hbm, y_hbm,
                  idxv, g0, g1, s0, s1, gs0, gs1, ss0, ss1):
    wid = lax.axis_index("s") * NC + lax.axis_index("c")
    tb0 = wid * _TPW
    pltpu.sync_copy(pos_hbm.at[pl.ds(tb0 * K, _TPW * K)], idxv)
    gb, sb = (g0, g1), (s0, s1)
    gsem, ssem = (gs0, gs1), (ss0, ss1)
    gcop, scop = [], []

    def combine_chunk(c):
        gcop[c].wait()
        scop[c].wait()
        gv, sv = gb[c % 2], sb[c % 2]
        for r in range(_CTOK):
            def body(c2, _, r=r, gv=gv, sv=sv):
                off = c2 * 16
                yv = (gv[2 * r, pl.ds(off, 16)]
                      + gv[2 * r + 1, pl.ds(off, 16)]
                      + sv[r, pl.ds(off, 16)])
                sv[r, pl.ds(off, 16)] = yv
                return 0
            lax.fori_loop(0, D // 16, body, 0)
        pltpu.sync_copy(sv, y_hbm.at[pl.ds(tb0 + c * _CTOK, _CTOK)])

    for ch in range(_FNC):
        gcop.append(pltpu.async_copy(
            ys_hbm.at[idxv.at[pl.ds(ch * _CTOK * K, _CTOK * K)]],
            gb[ch % 2], gsem[ch % 2]))
        scop.append(pltpu.async_copy(
            s_hbm.at[pl.ds(tb0 + ch * _CTOK, _CTOK)],
            sb[ch % 2], ssem[ch % 2]))
        if ch > 0:
            combine_chunk(ch - 1)
    combine_chunk(_FNC - 1)


def _combine(posf, ys, s):
    mesh = plsc.VectorSubcoreMesh(core_axis_name="c", subcore_axis_name="s")
    return pl.kernel(
        _combine_body,
        out_type=jax.ShapeDtypeStruct((T, D), jnp.float32),
        mesh=mesh,
        compiler_params=pltpu.CompilerParams(needs_layout_passes=False),
        scratch_types=[
            pltpu.VMEM((_TPW * K,), jnp.int32),
            pltpu.VMEM((_CTOK * K, D), jnp.float32),
            pltpu.VMEM((_CTOK * K, D), jnp.float32),
            pltpu.VMEM((_CTOK, D), jnp.float32),
            pltpu.VMEM((_CTOK, D), jnp.float32),
            pltpu.SemaphoreType.DMA,
            pltpu.SemaphoreType.DMA,
            pltpu.SemaphoreType.DMA,
            pltpu.SemaphoreType.DMA,
        ],
    )(posf, ys, s)


def kernel(x, gate_w, W1, b1, W2, b2, W3, b3, Ws1, bs1, Ws2, bs2, Ws3, bs3):
    shape = x.shape
    xf = x.reshape(-1, shape[-1])

    pos, wpair, texp2 = _route(xf, gate_w)
    texp = texp2.reshape(NT)
    posf = pos.reshape(T * K)
    wf = wpair.reshape(T * K)

    xpack = lax.bitcast_convert_type(
        xf.astype(jnp.bfloat16).reshape(T, DP, 2), jnp.uint32)
    src, wrow = _scatter(posf, wf)
    xs_pack = _gather(src, xpack)
    xsb = lax.bitcast_convert_type(xs_pack, jnp.bfloat16).reshape(NP, D)
    ys = _grouped_ffn(texp, xsb, wrow.reshape(NP, 1),
                      W1, b1.reshape(E, 1, F), W2, b2.reshape(E, 1, D),
                      W3, b3.reshape(E, 1, F))
    s = _shared_ffn(xf, Ws1.astype(jnp.bfloat16), bs1,
                    Ws2.astype(jnp.bfloat16), bs2,
                    Ws3.astype(jnp.bfloat16), bs3)
    y = _combine(posf, ys, s)
    return y.reshape(shape)


# speedup vs baseline: 1.5163x; 1.5133x over previous
import functools

import jax
import jax.numpy as jnp
from jax import lax
from jax.experimental import pallas as pl
from jax.experimental.pallas import tpu as pltpu
from jax.experimental.pallas import tpu_sc as plsc

T = 2048
D = 1024
F = 256
E = 64
K = 2
FS = 2048
TM = 64
NT = (T * K) // TM + E
NP = NT * TM
DP = D // 2

NC, NS = 2, 16
NW = NC * NS


def _cumsum_rows(a):
    n = a.shape[0]
    s = 1
    while s < n:
        a = a + jnp.concatenate(
            [jnp.zeros((s, a.shape[1]), a.dtype), a[:-s]], axis=0)
        s *= 2
    return a


def _route_body(x_ref, gw_ref, pos_ref, w_ref, texp_ref, xp_ref):
    xf = x_ref[...]
    gw = gw_ref[...]
    logits = lax.dot_general(xf, gw, (((1,), (1,)), ((), ())),
                             preferred_element_type=jnp.float32)
    m = jnp.max(logits, axis=-1, keepdims=True)
    ex = jnp.exp(logits - m)
    sc = ex / jnp.sum(ex, axis=-1, keepdims=True)

    lane = lax.broadcasted_iota(jnp.int32, (T, E), 1)
    m1 = jnp.max(sc, axis=-1, keepdims=True)
    i1 = jnp.min(jnp.where(sc == m1, lane, E), axis=-1, keepdims=True)
    sc2 = jnp.where(lane == i1, -1.0, sc)
    m2 = jnp.max(sc2, axis=-1, keepdims=True)
    i2 = jnp.min(jnp.where(sc2 == m2, lane, E), axis=-1, keepdims=True)

    d = jnp.exp(m2 - m1)
    w1 = 1.0 / (1.0 + d)
    w2 = d / (1.0 + d)

    oh1 = (lane == i1).astype(jnp.float32)
    oh2 = (lane == i2).astype(jnp.float32)
    cum1 = _cumsum_rows(oh1)
    cum2 = _cumsum_rows(oh2)
    c1 = cum1[T - 1:T, :]
    counts = c1 + cum2[T - 1:T, :]

    rank1 = jnp.sum(oh1 * cum1, axis=-1, keepdims=True) - 1.0
    rank2 = jnp.sum(oh2 * (cum2 + c1), axis=-1, keepdims=True) - 1.0

    nt = (counts.astype(jnp.int32) + (TM - 1)) // TM
    a64 = lax.broadcasted_iota(jnp.int32, (E, E), 0)
    b64 = lax.broadcasted_iota(jnp.int32, (E, E), 1)
    tri = (a64 < b64).astype(jnp.float32)
    po_f = lax.dot_general(nt.astype(jnp.float32), tri,
                           (((1,), (0,)), ((), ())),
                           preferred_element_type=jnp.float32) * TM

    pos1 = (jnp.sum(oh1 * po_f, axis=-1, keepdims=True)
            + rank1).astype(jnp.int32)
    pos2 = (jnp.sum(oh2 * po_f, axis=-1, keepdims=True)
            + rank2).astype(jnp.int32)

    pos_ref[...] = jnp.concatenate([pos1, pos2], axis=1)
    w_ref[...] = jnp.concatenate([w1, w2], axis=1)

    u = lax.bitcast_convert_type(xf, jnp.uint32)
    r = (u + 0x7FFF + ((u >> 16) & 1)) >> 16
    xp_ref[...] = r[:, :DP] | (r[:, DP:] << 16)

    jt = lax.broadcasted_iota(jnp.int32, (NT, E), 0)
    start = po_f.astype(jnp.int32) // TM
    ind = (jt >= start) & (jt < start + nt)
    lane2 = lax.broadcasted_iota(jnp.int32, (NT, E), 1)
    texp_ref[...] = jnp.sum(jnp.where(ind, lane2, 0), axis=1,
                            keepdims=True)


def _route(xf, gate_w):
    return pl.pallas_call(
        _route_body,
        out_shape=[
            jax.ShapeDtypeStruct((T, K), jnp.int32),
            jax.ShapeDtypeStruct((T, K), jnp.float32),
            jax.ShapeDtypeStruct((NT, 1), jnp.int32),
            jax.ShapeDtypeStruct((T, DP), jnp.uint32),
        ],
    )(xf, gate_w)


def _scatter_body(pos_hbm, wp_hbm, src_hbm, wrow_hbm,
                  posv, wpv, srcv, wrv, sem):
    wid = lax.axis_index("s") * NC + lax.axis_index("c")

    @pl.when(wid == 0)
    def _():
        pltpu.sync_copy(pos_hbm, posv)
        pltpu.sync_copy(wp_hbm, wpv)

        def init(i, _):
            srcv[pl.ds(i * 16, 16)] = jnp.zeros((16,), jnp.int32)
            wrv[pl.ds(i * 16, 16)] = jnp.zeros((16,), jnp.float32)
            return 0
        lax.fori_loop(0, NP // 16, init, 0)

        iota = lax.iota(jnp.int32, 16)

        def scat(i, _):
            off = i * 16
            idx = posv[pl.ds(off, 16)]
            tok = (off + iota) >> 1
            wv16 = wpv[pl.ds(off, 16)]
            plsc.store_scatter(srcv, [idx], tok)
            plsc.store_scatter(wrv, [idx], wv16)
            return 0
        lax.fori_loop(0, (T * K) // 16, scat, 0)

        pltpu.sync_copy(srcv, src_hbm)
        pltpu.sync_copy(wrv, wrow_hbm)


def _scatter(posf, wf):
    mesh = plsc.VectorSubcoreMesh(core_axis_name="c", subcore_axis_name="s")
    return pl.kernel(
        _scatter_body,
        out_type=[
            jax.ShapeDtypeStruct((NP,), jnp.int32),
            jax.ShapeDtypeStruct((NP,), jnp.float32),
        ],
        mesh=mesh,
        compiler_params=pltpu.CompilerParams(needs_layout_passes=False),
        scratch_types=[
            pltpu.VMEM((T * K,), jnp.int32),
            pltpu.VMEM((T * K,), jnp.float32),
            pltpu.VMEM((NP,), jnp.int32),
            pltpu.VMEM((NP,), jnp.float32),
            pltpu.SemaphoreType.DMA,
        ],
    )(posf, wf)


_GCH = 32
_GPW = NP // NW
_GNC = _GPW // _GCH
_GNB = 4


def _gather_body(src_hbm, x_hbm, xs_hbm, idxv, r0, r1, r2, r3,
                 s0, s1, s2, s3, osem):
    wid = lax.axis_index("s") * NC + lax.axis_index("c")
    base = wid * _GPW
    pltpu.sync_copy(src_hbm.at[pl.ds(base, _GPW)], idxv)
    bufs = (r0, r1, r2, r3)
    sems = (s0, s1, s2, s3)

    def fire(g):
        return pltpu.async_copy(
            x_hbm.at[idxv.at[pl.ds(g * _GCH, _GCH)]], bufs[g % _GNB],
            sems[g % _GNB])

    copies = [fire(g) for g in range(_GNB)]
    outs = []
    for g in range(_GNC):
        copies[g].wait()
        outs.append(pltpu.async_copy(
            bufs[g % _GNB], xs_hbm.at[pl.ds(base + g * _GCH, _GCH)], osem))
        if g + _GNB < _GNC:
            outs[g].wait()
            copies.append(fire(g + _GNB))
    for g in range(max(0, _GNC - _GNB), _GNC):
        outs[g].wait()


def _gather(src, xf):
    mesh = plsc.VectorSubcoreMesh(core_axis_name="c", subcore_axis_name="s")
    return pl.kernel(
        _gather_body,
        out_type=jax.ShapeDtypeStruct((NP, DP), jnp.uint32),
        mesh=mesh,
        compiler_params=pltpu.CompilerParams(needs_layout_passes=False),
        scratch_types=[
            pltpu.VMEM((_GPW,), jnp.int32),
            pltpu.VMEM((_GCH, DP), jnp.uint32),
            pltpu.VMEM((_GCH, DP), jnp.uint32),
            pltpu.VMEM((_GCH, DP), jnp.uint32),
            pltpu.VMEM((_GCH, DP), jnp.uint32),
            pltpu.SemaphoreType.DMA,
            pltpu.SemaphoreType.DMA,
            pltpu.SemaphoreType.DMA,
            pltpu.SemaphoreType.DMA,
            pltpu.SemaphoreType.DMA,
        ],
    )(src, xf)


def _ffn_body(texp_ref, xs_ref, w_ref, w1_ref, b1_ref, w2_ref, b2_ref,
              w3_ref, b3_ref, ys_ref):
    xw = xs_ref[...]
    x_lo = lax.bitcast_convert_type(xw << 16, jnp.float32)
    x_hi = lax.bitcast_convert_type(xw & jnp.uint32(0xFFFF0000), jnp.float32)

    def two_dot(wr):
        w = wr[0]
        return (lax.dot_general(x_lo, w[:, :DP], (((1,), (1,)), ((), ())),
                                preferred_element_type=jnp.float32)
                + lax.dot_general(x_hi, w[:, DP:], (((1,), (1,)), ((), ())),
                                  preferred_element_type=jnp.float32))

    h1 = two_dot(w1_ref) + b1_ref[0]
    h3 = two_dot(w3_ref) + b3_ref[0]
    p = h1 * h3
    h = p / (1.0 + jnp.exp(-p))
    out = lax.dot_general(h, w2_ref[0], (((1,), (1,)), ((), ())),
                          preferred_element_type=jnp.float32) + b2_ref[0]
    ys_ref[...] = out * w_ref[...]


def _grouped_ffn(texp, xs, wrow, W1, b1, W2, b2, W3, b3):
    grid_spec = pltpu.PrefetchScalarGridSpec(
        num_scalar_prefetch=1,
        grid=(NT,),
        in_specs=[
            pl.BlockSpec((TM, DP), lambda j, te: (j, 0)),
            pl.BlockSpec((TM, 1), lambda j, te: (j, 0)),
            pl.BlockSpec((1, F, D), lambda j, te: (te[j], 0, 0)),
            pl.BlockSpec((1, 1, F), lambda j, te: (te[j], 0, 0)),
            pl.BlockSpec((1, D, F), lambda j, te: (te[j], 0, 0)),
            pl.BlockSpec((1, 1, D), lambda j, te: (te[j], 0, 0)),
            pl.BlockSpec((1, F, D), lambda j, te: (te[j], 0, 0)),
            pl.BlockSpec((1, 1, F), lambda j, te: (te[j], 0, 0)),
        ],
        out_specs=pl.BlockSpec((TM, D), lambda j, te: (j, 0)),
    )
    return pl.pallas_call(
        _ffn_body,
        grid_spec=grid_spec,
        out_shape=jax.ShapeDtypeStruct((NP, D), jnp.float32),
    )(texp, xs, wrow, W1, b1, W2, b2, W3, b3)


def _shared_body(x_ref, w1_ref, b1_ref, w2_ref, b2_ref, w3_ref, b3_ref,
                 s_ref):
    xt = x_ref[...]
    h1 = lax.dot_general(xt, w1_ref[...], (((1,), (1,)), ((), ())),
                         preferred_element_type=jnp.float32) + b1_ref[...]
    h3 = lax.dot_general(xt, w3_ref[...], (((1,), (1,)), ((), ())),
                         preferred_element_type=jnp.float32) + b3_ref[...]
    p = h1 * h3
    h = p / (1.0 + jnp.exp(-p))
    s_ref[...] = lax.dot_general(h, w2_ref[...], (((1,), (1,)), ((), ())),
                                 preferred_element_type=jnp.float32) \
        + b2_ref[...]


_ET = 128


def _shared_ffn(xf, Ws1, bs1, Ws2, bs2, Ws3, bs3):
    return pl.pallas_call(
        _shared_body,
        grid=(T // _ET,),
        in_specs=[
            pl.BlockSpec((_ET, D), lambda i: (i, 0)),
            pl.BlockSpec((FS, D), lambda i: (0, 0)),
            pl.BlockSpec((FS,), lambda i: (0,)),
            pl.BlockSpec((D, FS), lambda i: (0, 0)),
            pl.BlockSpec((D,), lambda i: (0,)),
            pl.BlockSpec((FS, D), lambda i: (0, 0)),
            pl.BlockSpec((FS,), lambda i: (0,)),
        ],
        out_specs=pl.BlockSpec((_ET, D), lambda i: (i, 0)),
        out_shape=jax.ShapeDtypeStruct((T, D), jnp.float32),
    )(xf, Ws1, bs1, Ws2, bs2, Ws3, bs3)


_CTOK = 8
_TPW = T // NW


_FNC = _TPW // _CTOK


def _combine_body(pos_hbm, ys_hbm, s_hbm, y_hbm,
                  idxv, g0, g1, s0, s1, gs0, gs1, ss0, ss1):
    wid = lax.axis_index("s") * NC + lax.axis_index("c")
    tb0 = wid * _TPW
    pltpu.sync_copy(pos_hbm.at[pl.ds(tb0 * K, _TPW * K)], idxv)
    gb, sb = (g0, g1), (s0, s1)
    gsem, ssem = (gs0, gs1), (ss0, ss1)
    gcop, scop = [], []

    def combine_chunk(c):
        gcop[c].wait()
        scop[c].wait()
        gv, sv = gb[c % 2], sb[c % 2]
        for r in range(_CTOK):
            def body(c2, _, r=r, gv=gv, sv=sv):
                off = c2 * 16
                yv = (gv[2 * r, pl.ds(off, 16)]
                      + gv[2 * r + 1, pl.ds(off, 16)]
                      + sv[r, pl.ds(off, 16)])
                sv[r, pl.ds(off, 16)] = yv
                return 0
            lax.fori_loop(0, D // 16, body, 0)
        pltpu.sync_copy(sv, y_hbm.at[pl.ds(tb0 + c * _CTOK, _CTOK)])

    for ch in range(_FNC):
        gcop.append(pltpu.async_copy(
            ys_hbm.at[idxv.at[pl.ds(ch * _CTOK * K, _CTOK * K)]],
            gb[ch % 2], gsem[ch % 2]))
        scop.append(pltpu.async_copy(
            s_hbm.at[pl.ds(tb0 + ch * _CTOK, _CTOK)],
            sb[ch % 2], ssem[ch % 2]))
        if ch > 0:
            combine_chunk(ch - 1)
    combine_chunk(_FNC - 1)


def _combine(posf, ys, s):
    mesh = plsc.VectorSubcoreMesh(core_axis_name="c", subcore_axis_name="s")
    return pl.kernel(
        _combine_body,
        out_type=jax.ShapeDtypeStruct((T, D), jnp.float32),
        mesh=mesh,
        compiler_params=pltpu.CompilerParams(needs_layout_passes=False),
        scratch_types=[
            pltpu.VMEM((_TPW * K,), jnp.int32),
            pltpu.VMEM((_CTOK * K, D), jnp.float32),
            pltpu.VMEM((_CTOK * K, D), jnp.float32),
            pltpu.VMEM((_CTOK, D), jnp.float32),
            pltpu.VMEM((_CTOK, D), jnp.float32),
            pltpu.SemaphoreType.DMA,
            pltpu.SemaphoreType.DMA,
            pltpu.SemaphoreType.DMA,
            pltpu.SemaphoreType.DMA,
        ],
    )(posf, ys, s)


def kernel(x, gate_w, W1, b1, W2, b2, W3, b3, Ws1, bs1, Ws2, bs2, Ws3, bs3):
    shape = x.shape
    xf = x.reshape(-1, shape[-1])

    pos, wpair, texp2, xpack = _route(xf, gate_w)
    texp = texp2.reshape(NT)
    posf = pos.reshape(T * K)
    wf = wpair.reshape(T * K)

    srcv, wrow = _scatter(posf, wf)
    xs = _gather(srcv, xpack)
    ys = _grouped_ffn(texp, xs, wrow.reshape(NP, 1),
                      W1, b1.reshape(E, 1, F), W2, b2.reshape(E, 1, D),
                      W3, b3.reshape(E, 1, F))
    s = _shared_ffn(xf, Ws1, bs1, Ws2, bs2, Ws3, bs3)
    y = _combine(posf, ys, s)
    return y.reshape(shape)


# speedup vs baseline: 1.5392x; 1.0151x over previous
import functools

import jax
import jax.numpy as jnp
from jax import lax
from jax.experimental import pallas as pl
from jax.experimental.pallas import tpu as pltpu
from jax.experimental.pallas import tpu_sc as plsc

T = 2048
D = 1024
F = 256
E = 64
K = 2
FS = 2048
TM = 64
NT = (T * K) // TM + E
NP = NT * TM
DP = D // 2

NC, NS = 2, 16
NW = NC * NS


def _cumsum_rows(a):
    n = a.shape[0]
    s = 1
    while s < n:
        a = a + jnp.concatenate(
            [jnp.zeros((s, a.shape[1]), a.dtype), a[:-s]], axis=0)
        s *= 2
    return a


def _route_body(x_ref, gw_ref, pos_ref, w_ref, texp_ref, xp_ref):
    xf = x_ref[...]
    gw = gw_ref[...]
    logits = lax.dot_general(xf, gw, (((1,), (1,)), ((), ())),
                             preferred_element_type=jnp.float32)
    m = jnp.max(logits, axis=-1, keepdims=True)
    ex = jnp.exp(logits - m)
    sc = ex / jnp.sum(ex, axis=-1, keepdims=True)

    lane = lax.broadcasted_iota(jnp.int32, (T, E), 1)
    m1 = jnp.max(sc, axis=-1, keepdims=True)
    i1 = jnp.min(jnp.where(sc == m1, lane, E), axis=-1, keepdims=True)
    sc2 = jnp.where(lane == i1, -1.0, sc)
    m2 = jnp.max(sc2, axis=-1, keepdims=True)
    i2 = jnp.min(jnp.where(sc2 == m2, lane, E), axis=-1, keepdims=True)

    d = jnp.exp(m2 - m1)
    w1 = 1.0 / (1.0 + d)
    w2 = d / (1.0 + d)

    oh1 = (lane == i1).astype(jnp.float32)
    oh2 = (lane == i2).astype(jnp.float32)
    cum1 = _cumsum_rows(oh1)
    cum2 = _cumsum_rows(oh2)
    c1 = cum1[T - 1:T, :]
    counts = c1 + cum2[T - 1:T, :]

    rank1 = jnp.sum(oh1 * cum1, axis=-1, keepdims=True) - 1.0
    rank2 = jnp.sum(oh2 * (cum2 + c1), axis=-1, keepdims=True) - 1.0

    nt = (counts.astype(jnp.int32) + (TM - 1)) // TM
    a64 = lax.broadcasted_iota(jnp.int32, (E, E), 0)
    b64 = lax.broadcasted_iota(jnp.int32, (E, E), 1)
    tri = (a64 < b64).astype(jnp.float32)
    po_f = lax.dot_general(nt.astype(jnp.float32), tri,
                           (((1,), (0,)), ((), ())),
                           preferred_element_type=jnp.float32) * TM

    pos1 = (jnp.sum(oh1 * po_f, axis=-1, keepdims=True)
            + rank1).astype(jnp.int32)
    pos2 = (jnp.sum(oh2 * po_f, axis=-1, keepdims=True)
            + rank2).astype(jnp.int32)

    pos_ref[...] = jnp.concatenate([pos1, pos2], axis=1)
    w_ref[...] = jnp.concatenate([w1, w2], axis=1)

    u = lax.bitcast_convert_type(xf, jnp.uint32)
    r = (u + 0x7FFF + ((u >> 16) & 1)) >> 16
    xp_ref[...] = r[:, :DP] | (r[:, DP:] << 16)

    jt = lax.broadcasted_iota(jnp.int32, (NT, E), 0)
    start = po_f.astype(jnp.int32) // TM
    ind = (jt >= start) & (jt < start + nt)
    lane2 = lax.broadcasted_iota(jnp.int32, (NT, E), 1)
    texp_ref[...] = jnp.sum(jnp.where(ind, lane2, 0), axis=1,
                            keepdims=True)


def _route(xf, gate_w):
    return pl.pallas_call(
        _route_body,
        out_shape=[
            jax.ShapeDtypeStruct((T, K), jnp.int32),
            jax.ShapeDtypeStruct((T, K), jnp.float32),
            jax.ShapeDtypeStruct((NT, 1), jnp.int32),
            jax.ShapeDtypeStruct((T, DP), jnp.uint32),
        ],
    )(xf, gate_w)


def _scatter_body(pos_hbm, wp_hbm, src_hbm, wrow_hbm,
                  posv, wpv, srcv, wrv, sem):
    wid = lax.axis_index("s") * NC + lax.axis_index("c")

    @pl.when(wid == 0)
    def _():
        pltpu.sync_copy(pos_hbm, posv)
        pltpu.sync_copy(wp_hbm, wpv)

        def init(i, _):
            srcv[pl.ds(i * 16, 16)] = jnp.zeros((16,), jnp.int32)
            wrv[pl.ds(i * 16, 16)] = jnp.zeros((16,), jnp.float32)
            return 0
        lax.fori_loop(0, NP // 16, init, 0)

        iota = lax.iota(jnp.int32, 16)

        def scat(i, _):
            off = i * 16
            idx = posv[pl.ds(off, 16)]
            tok = (off + iota) >> 1
            wv16 = wpv[pl.ds(off, 16)]
            plsc.store_scatter(srcv, [idx], tok)
            plsc.store_scatter(wrv, [idx], wv16)
            return 0
        lax.fori_loop(0, (T * K) // 16, scat, 0)

        pltpu.sync_copy(srcv, src_hbm)
        pltpu.sync_copy(wrv, wrow_hbm)


def _scatter(posf, wf):
    mesh = plsc.VectorSubcoreMesh(core_axis_name="c", subcore_axis_name="s")
    return pl.kernel(
        _scatter_body,
        out_type=[
            jax.ShapeDtypeStruct((NP,), jnp.int32),
            jax.ShapeDtypeStruct((NP,), jnp.float32),
        ],
        mesh=mesh,
        compiler_params=pltpu.CompilerParams(needs_layout_passes=False),
        scratch_types=[
            pltpu.VMEM((T * K,), jnp.int32),
            pltpu.VMEM((T * K,), jnp.float32),
            pltpu.VMEM((NP,), jnp.int32),
            pltpu.VMEM((NP,), jnp.float32),
            pltpu.SemaphoreType.DMA,
        ],
    )(posf, wf)


_GCH = 32
NPH = NP // 2
_GPW = NPH // NW
_GNC = _GPW // _GCH
_GNB = 4


def _make_gather_body(off):
  def _gather_body(src_hbm, x_hbm, xs_hbm, idxv, r0, r1, r2, r3,
                   s0, s1, s2, s3, osem):
    wid = lax.axis_index("s") * NC + lax.axis_index("c")
    base = wid * _GPW
    pltpu.sync_copy(src_hbm.at[pl.ds(off + base, _GPW)], idxv)
    bufs = (r0, r1, r2, r3)
    sems = (s0, s1, s2, s3)

    def fire(g):
        return pltpu.async_copy(
            x_hbm.at[idxv.at[pl.ds(g * _GCH, _GCH)]], bufs[g % _GNB],
            sems[g % _GNB])

    copies = [fire(g) for g in range(_GNB)]
    outs = []
    for g in range(_GNC):
        copies[g].wait()
        outs.append(pltpu.async_copy(
            bufs[g % _GNB], xs_hbm.at[pl.ds(base + g * _GCH, _GCH)], osem))
        if g + _GNB < _GNC:
            outs[g].wait()
            copies.append(fire(g + _GNB))
    for g in range(max(0, _GNC - _GNB), _GNC):
        outs[g].wait()
  return _gather_body


def _gather(src, xf, off):
    mesh = plsc.VectorSubcoreMesh(core_axis_name="c", subcore_axis_name="s")
    return pl.kernel(
        _make_gather_body(off),
        out_type=jax.ShapeDtypeStruct((NPH, DP), jnp.uint32),
        mesh=mesh,
        compiler_params=pltpu.CompilerParams(needs_layout_passes=False),
        scratch_types=[
            pltpu.VMEM((_GPW,), jnp.int32),
            pltpu.VMEM((_GCH, DP), jnp.uint32),
            pltpu.VMEM((_GCH, DP), jnp.uint32),
            pltpu.VMEM((_GCH, DP), jnp.uint32),
            pltpu.VMEM((_GCH, DP), jnp.uint32),
            pltpu.SemaphoreType.DMA,
            pltpu.SemaphoreType.DMA,
            pltpu.SemaphoreType.DMA,
            pltpu.SemaphoreType.DMA,
            pltpu.SemaphoreType.DMA,
        ],
    )(src, xf)


def _ffn_body(texp_ref, xs_ref, w_ref, w1_ref, b1_ref, w2_ref, b2_ref,
              w3_ref, b3_ref, ys_ref):
    xw = xs_ref[...]
    x_lo = lax.bitcast_convert_type(xw << 16, jnp.float32)
    x_hi = lax.bitcast_convert_type(xw & jnp.uint32(0xFFFF0000), jnp.float32)

    def two_dot(wr):
        w = wr[0]
        return (lax.dot_general(x_lo, w[:, :DP], (((1,), (1,)), ((), ())),
                                preferred_element_type=jnp.float32)
                + lax.dot_general(x_hi, w[:, DP:], (((1,), (1,)), ((), ())),
                                  preferred_element_type=jnp.float32))

    h1 = two_dot(w1_ref) + b1_ref[0]
    h3 = two_dot(w3_ref) + b3_ref[0]
    p = h1 * h3
    h = p / (1.0 + jnp.exp(-p))
    out = lax.dot_general(h, w2_ref[0], (((1,), (1,)), ((), ())),
                          preferred_element_type=jnp.float32) + b2_ref[0]
    ys_ref[...] = out * w_ref[...]


def _ffn_body_alias(texp_ref, xs_ref, w_ref, w1_ref, b1_ref, w2_ref,
                    b2_ref, w3_ref, b3_ref, ysin_ref, ys_ref):
    _ffn_body(texp_ref, xs_ref, w_ref, w1_ref, b1_ref, w2_ref, b2_ref,
              w3_ref, b3_ref, ys_ref)


NTH = NT // 2


def _grouped_ffn_half1(texp, xs1, wrow, W1, b1, W2, b2, W3, b3):
    grid_spec = pltpu.PrefetchScalarGridSpec(
        num_scalar_prefetch=1,
        grid=(NTH,),
        in_specs=[
            pl.BlockSpec((TM, DP), lambda j, te: (j, 0)),
            pl.BlockSpec((TM, 1), lambda j, te: (j, 0)),
            pl.BlockSpec((1, F, D), lambda j, te: (te[j], 0, 0)),
            pl.BlockSpec((1, 1, F), lambda j, te: (te[j], 0, 0)),
            pl.BlockSpec((1, D, F), lambda j, te: (te[j], 0, 0)),
            pl.BlockSpec((1, 1, D), lambda j, te: (te[j], 0, 0)),
            pl.BlockSpec((1, F, D), lambda j, te: (te[j], 0, 0)),
            pl.BlockSpec((1, 1, F), lambda j, te: (te[j], 0, 0)),
        ],
        out_specs=pl.BlockSpec((TM, D), lambda j, te: (j, 0)),
    )
    return pl.pallas_call(
        _ffn_body,
        grid_spec=grid_spec,
        out_shape=jax.ShapeDtypeStruct((NP, D), jnp.float32),
    )(texp, xs1, wrow, W1, b1, W2, b2, W3, b3)


def _grouped_ffn_half2(texp, xs2, wrow, W1, b1, W2, b2, W3, b3, ys1):
    grid_spec = pltpu.PrefetchScalarGridSpec(
        num_scalar_prefetch=1,
        grid=(NTH,),
        in_specs=[
            pl.BlockSpec((TM, DP), lambda j, te: (j, 0)),
            pl.BlockSpec((TM, 1), lambda j, te: (j + NTH, 0)),
            pl.BlockSpec((1, F, D), lambda j, te: (te[j + NTH], 0, 0)),
            pl.BlockSpec((1, 1, F), lambda j, te: (te[j + NTH], 0, 0)),
            pl.BlockSpec((1, D, F), lambda j, te: (te[j + NTH], 0, 0)),
            pl.BlockSpec((1, 1, D), lambda j, te: (te[j + NTH], 0, 0)),
            pl.BlockSpec((1, F, D), lambda j, te: (te[j + NTH], 0, 0)),
            pl.BlockSpec((1, 1, F), lambda j, te: (te[j + NTH], 0, 0)),
            pl.BlockSpec((TM, D), lambda j, te: (j + NTH, 0)),
        ],
        out_specs=pl.BlockSpec((TM, D), lambda j, te: (j + NTH, 0)),
    )
    return pl.pallas_call(
        _ffn_body_alias,
        grid_spec=grid_spec,
        out_shape=jax.ShapeDtypeStruct((NP, D), jnp.float32),
        input_output_aliases={9: 0},
    )(texp, xs2, wrow, W1, b1, W2, b2, W3, b3, ys1)


def _shared_body(x_ref, w1_ref, b1_ref, w2_ref, b2_ref, w3_ref, b3_ref,
                 s_ref):
    xt = x_ref[...]
    h1 = lax.dot_general(xt, w1_ref[...], (((1,), (1,)), ((), ())),
                         preferred_element_type=jnp.float32) + b1_ref[...]
    h3 = lax.dot_general(xt, w3_ref[...], (((1,), (1,)), ((), ())),
                         preferred_element_type=jnp.float32) + b3_ref[...]
    p = h1 * h3
    h = p / (1.0 + jnp.exp(-p))
    s_ref[...] = lax.dot_general(h, w2_ref[...], (((1,), (1,)), ((), ())),
                                 preferred_element_type=jnp.float32) \
        + b2_ref[...]


_ET = 128


def _shared_ffn(xf, Ws1, bs1, Ws2, bs2, Ws3, bs3):
    return pl.pallas_call(
        _shared_body,
        grid=(T // _ET,),
        in_specs=[
            pl.BlockSpec((_ET, D), lambda i: (i, 0)),
            pl.BlockSpec((FS, D), lambda i: (0, 0)),
            pl.BlockSpec((FS,), lambda i: (0,)),
            pl.BlockSpec((D, FS), lambda i: (0, 0)),
            pl.BlockSpec((D,), lambda i: (0,)),
            pl.BlockSpec((FS, D), lambda i: (0, 0)),
            pl.BlockSpec((FS,), lambda i: (0,)),
        ],
        out_specs=pl.BlockSpec((_ET, D), lambda i: (i, 0)),
        out_shape=jax.ShapeDtypeStruct((T, D), jnp.float32),
    )(xf, Ws1, bs1, Ws2, bs2, Ws3, bs3)


_CTOK = 8
_TPW = T // NW


_FNC = _TPW // _CTOK


def _combine_body(pos_hbm, ys_hbm, s_hbm, y_hbm,
                  idxv, g0, g1, s0, s1, gs0, gs1, ss0, ss1):
    wid = lax.axis_index("s") * NC + lax.axis_index("c")
    tb0 = wid * _TPW
    pltpu.sync_copy(pos_hbm.at[pl.ds(tb0 * K, _TPW * K)], idxv)
    gb, sb = (g0, g1), (s0, s1)
    gsem, ssem = (gs0, gs1), (ss0, ss1)
    gcop, scop = [], []

    def combine_chunk(c):
        gcop[c].wait()
        scop[c].wait()
        gv, sv = gb[c % 2], sb[c % 2]
        for r in range(_CTOK):
            def body(c2, _, r=r, gv=gv, sv=sv):
                off = c2 * 16
                yv = (gv[2 * r, pl.ds(off, 16)]
                      + gv[2 * r + 1, pl.ds(off, 16)]
                      + sv[r, pl.ds(off, 16)])
                sv[r, pl.ds(off, 16)] = yv
                return 0
            lax.fori_loop(0, D // 16, body, 0)
        pltpu.sync_copy(sv, y_hbm.at[pl.ds(tb0 + c * _CTOK, _CTOK)])

    for ch in range(_FNC):
        gcop.append(pltpu.async_copy(
            ys_hbm.at[idxv.at[pl.ds(ch * _CTOK * K, _CTOK * K)]],
            gb[ch % 2], gsem[ch % 2]))
        scop.append(pltpu.async_copy(
            s_hbm.at[pl.ds(tb0 + ch * _CTOK, _CTOK)],
            sb[ch % 2], ssem[ch % 2]))
        if ch > 0:
            combine_chunk(ch - 1)
    combine_chunk(_FNC - 1)


def _combine(posf, ys, s):
    mesh = plsc.VectorSubcoreMesh(core_axis_name="c", subcore_axis_name="s")
    return pl.kernel(
        _combine_body,
        out_type=jax.ShapeDtypeStruct((T, D), jnp.float32),
        mesh=mesh,
        compiler_params=pltpu.CompilerParams(needs_layout_passes=False),
        scratch_types=[
            pltpu.VMEM((_TPW * K,), jnp.int32),
            pltpu.VMEM((_CTOK * K, D), jnp.float32),
            pltpu.VMEM((_CTOK * K, D), jnp.float32),
            pltpu.VMEM((_CTOK, D), jnp.float32),
            pltpu.VMEM((_CTOK, D), jnp.float32),
            pltpu.SemaphoreType.DMA,
            pltpu.SemaphoreType.DMA,
            pltpu.SemaphoreType.DMA,
            pltpu.SemaphoreType.DMA,
        ],
    )(posf, ys, s)


def kernel(x, gate_w, W1, b1, W2, b2, W3, b3, Ws1, bs1, Ws2, bs2, Ws3, bs3):
    shape = x.shape
    xf = x.reshape(-1, shape[-1])

    pos, wpair, texp2, xpack = _route(xf, gate_w)
    texp = texp2.reshape(NT)
    posf = pos.reshape(T * K)
    wf = wpair.reshape(T * K)

    srcv, wrow = _scatter(posf, wf)
    wrow2 = wrow.reshape(NP, 1)
    b1r, b2r, b3r = b1.reshape(E, 1, F), b2.reshape(E, 1, D), b3.reshape(E, 1, F)
    xs1 = _gather(srcv, xpack, 0)
    xs2 = _gather(srcv, xpack, NPH)
    ys1 = _grouped_ffn_half1(texp, xs1, wrow2, W1, b1r, W2, b2r, W3, b3r)
    ys = _grouped_ffn_half2(texp, xs2, wrow2, W1, b1r, W2, b2r, W3, b3r, ys1)
    s = _shared_ffn(xf, Ws1, bs1, Ws2, bs2, Ws3, bs3)
    y = _combine(posf, ys, s)
    return y.reshape(shape)


# speedup vs baseline: 1.5460x; 1.0044x over previous
import jax
import jax.numpy as jnp
from jax import lax
from jax.experimental import pallas as pl
from jax.experimental.pallas import tpu as pltpu
from jax.experimental.pallas import tpu_sc as plsc

T = 2048
D = 1024
F = 256
E = 64
K = 2
FS = 2048
TM = 64
NT = (T * K) // TM + E
NP = NT * TM
DP = D // 2

NC, NS = 2, 16
NW = NC * NS


def _cumsum_rows(a):
    n = a.shape[0]
    s = 1
    while s < n:
        a = a + jnp.concatenate(
            [jnp.zeros((s, a.shape[1]), a.dtype), a[:-s]], axis=0)
        s *= 2
    return a


def _route_body(x_ref, gw_ref, pos_ref, w_ref, texp_ref, xp_ref):
    xf = x_ref[...]
    gw = gw_ref[...]
    logits = lax.dot_general(xf, gw, (((1,), (1,)), ((), ())),
                             preferred_element_type=jnp.float32)
    m = jnp.max(logits, axis=-1, keepdims=True)
    ex = jnp.exp(logits - m)
    sc = ex / jnp.sum(ex, axis=-1, keepdims=True)

    lane = lax.broadcasted_iota(jnp.int32, (T, E), 1)
    m1 = jnp.max(sc, axis=-1, keepdims=True)
    i1 = jnp.min(jnp.where(sc == m1, lane, E), axis=-1, keepdims=True)
    sc2 = jnp.where(lane == i1, -1.0, sc)
    m2 = jnp.max(sc2, axis=-1, keepdims=True)
    i2 = jnp.min(jnp.where(sc2 == m2, lane, E), axis=-1, keepdims=True)

    d = jnp.exp(m2 - m1)
    w1 = 1.0 / (1.0 + d)
    w2 = d / (1.0 + d)

    oh1 = (lane == i1).astype(jnp.float32)
    oh2 = (lane == i2).astype(jnp.float32)
    cum1 = _cumsum_rows(oh1)
    cum2 = _cumsum_rows(oh2)
    c1 = cum1[T - 1:T, :]
    counts = c1 + cum2[T - 1:T, :]

    rank1 = jnp.sum(oh1 * cum1, axis=-1, keepdims=True) - 1.0
    rank2 = jnp.sum(oh2 * (cum2 + c1), axis=-1, keepdims=True) - 1.0

    nt = (counts.astype(jnp.int32) + (TM - 1)) // TM
    a64 = lax.broadcasted_iota(jnp.int32, (E, E), 0)
    b64 = lax.broadcasted_iota(jnp.int32, (E, E), 1)
    tri = (a64 < b64).astype(jnp.float32)
    po_f = lax.dot_general(nt.astype(jnp.float32), tri,
                           (((1,), (0,)), ((), ())),
                           preferred_element_type=jnp.float32) * TM

    pos1 = (jnp.sum(oh1 * po_f, axis=-1, keepdims=True)
            + rank1).astype(jnp.int32)
    pos2 = (jnp.sum(oh2 * po_f, axis=-1, keepdims=True)
            + rank2).astype(jnp.int32)

    pos_ref[...] = jnp.concatenate([pos1, pos2], axis=1)
    w_ref[...] = jnp.concatenate([w1, w2], axis=1)

    u = lax.bitcast_convert_type(xf, jnp.uint32)
    r = (u + 0x7FFF + ((u >> 16) & 1)) >> 16
    xp_ref[...] = r[:, :DP] | (r[:, DP:] << 16)

    jt = lax.broadcasted_iota(jnp.int32, (NT, E), 0)
    start = po_f.astype(jnp.int32) // TM
    ind = (jt >= start) & (jt < start + nt)
    lane2 = lax.broadcasted_iota(jnp.int32, (NT, E), 1)
    texp_ref[...] = jnp.sum(jnp.where(ind, lane2, 0), axis=1,
                            keepdims=True)


def _route(xf, gate_w):
    return pl.pallas_call(
        _route_body,
        out_shape=[
            jax.ShapeDtypeStruct((T, K), jnp.int32),
            jax.ShapeDtypeStruct((T, K), jnp.float32),
            jax.ShapeDtypeStruct((NT, 1), jnp.int32),
            jax.ShapeDtypeStruct((T, DP), jnp.uint32),
        ],
    )(xf, gate_w)


def _scatter_body(pos_hbm, wp_hbm, src_hbm, wrow_hbm,
                  posv, wpv, srcv, wrv, sem):
    wid = lax.axis_index("s") * NC + lax.axis_index("c")

    @pl.when(wid == 0)
    def _():
        pltpu.sync_copy(pos_hbm, posv)
        pltpu.sync_copy(wp_hbm, wpv)

        def init(i, _):
            srcv[pl.ds(i * 16, 16)] = jnp.zeros((16,), jnp.int32)
            wrv[pl.ds(i * 16, 16)] = jnp.zeros((16,), jnp.float32)
            return 0
        lax.fori_loop(0, NP // 16, init, 0)

        iota = lax.iota(jnp.int32, 16)

        def scat(i, _):
            off = i * 16
            idx = posv[pl.ds(off, 16)]
            tok = (off + iota) >> 1
            wv16 = wpv[pl.ds(off, 16)]
            plsc.store_scatter(srcv, [idx], tok)
            plsc.store_scatter(wrv, [idx], wv16)
            return 0
        lax.fori_loop(0, (T * K) // 16, scat, 0)

        pltpu.sync_copy(srcv, src_hbm)
        pltpu.sync_copy(wrv, wrow_hbm)


def _scatter(posf, wf):
    mesh = plsc.VectorSubcoreMesh(core_axis_name="c", subcore_axis_name="s")
    return pl.kernel(
        _scatter_body,
        out_type=[
            jax.ShapeDtypeStruct((NP,), jnp.int32),
            jax.ShapeDtypeStruct((NP,), jnp.float32),
        ],
        mesh=mesh,
        compiler_params=pltpu.CompilerParams(needs_layout_passes=False),
        scratch_types=[
            pltpu.VMEM((T * K,), jnp.int32),
            pltpu.VMEM((T * K,), jnp.float32),
            pltpu.VMEM((NP,), jnp.int32),
            pltpu.VMEM((NP,), jnp.float32),
            pltpu.SemaphoreType.DMA,
        ],
    )(posf, wf)


_GCH = 32
NPH = NP // 2
_GPW = NPH // NW
_GNC = _GPW // _GCH
_GNB = 4


def _make_gather_body(off):
  def _gather_body(src_hbm, x_hbm, xs_hbm, idxv, r0, r1, r2, r3,
                   s0, s1, s2, s3, osem):
    wid = lax.axis_index("s") * NC + lax.axis_index("c")
    base = wid * _GPW
    pltpu.sync_copy(src_hbm.at[pl.ds(off + base, _GPW)], idxv)
    bufs = (r0, r1, r2, r3)
    sems = (s0, s1, s2, s3)

    def fire(g):
        return pltpu.async_copy(
            x_hbm.at[idxv.at[pl.ds(g * _GCH, _GCH)]], bufs[g % _GNB],
            sems[g % _GNB])

    copies = [fire(g) for g in range(_GNB)]
    outs = []
    for g in range(_GNC):
        copies[g].wait()
        outs.append(pltpu.async_copy(
            bufs[g % _GNB], xs_hbm.at[pl.ds(base + g * _GCH, _GCH)], osem))
        if g + _GNB < _GNC:
            outs[g].wait()
            copies.append(fire(g + _GNB))
    for g in range(max(0, _GNC - _GNB), _GNC):
        outs[g].wait()
  return _gather_body


def _gather(src, xf, off):
    mesh = plsc.VectorSubcoreMesh(core_axis_name="c", subcore_axis_name="s")
    return pl.kernel(
        _make_gather_body(off),
        out_type=jax.ShapeDtypeStruct((NPH, DP), jnp.uint32),
        mesh=mesh,
        compiler_params=pltpu.CompilerParams(needs_layout_passes=False),
        scratch_types=[
            pltpu.VMEM((_GPW,), jnp.int32),
            pltpu.VMEM((_GCH, DP), jnp.uint32),
            pltpu.VMEM((_GCH, DP), jnp.uint32),
            pltpu.VMEM((_GCH, DP), jnp.uint32),
            pltpu.VMEM((_GCH, DP), jnp.uint32),
            pltpu.SemaphoreType.DMA,
            pltpu.SemaphoreType.DMA,
            pltpu.SemaphoreType.DMA,
            pltpu.SemaphoreType.DMA,
            pltpu.SemaphoreType.DMA,
        ],
    )(src, xf)


def _ffn_body(texp_ref, xs_ref, w_ref, w1_ref, b1_ref, w2_ref, b2_ref,
              w3_ref, b3_ref, ys_ref):
    xw = xs_ref[...]
    x_lo = lax.bitcast_convert_type(xw << 16, jnp.float32)
    x_hi = lax.bitcast_convert_type(xw & jnp.uint32(0xFFFF0000), jnp.float32)

    def two_dot(wr):
        w = wr[0]
        return (lax.dot_general(x_lo, w[:, :DP], (((1,), (1,)), ((), ())),
                                preferred_element_type=jnp.float32)
                + lax.dot_general(x_hi, w[:, DP:], (((1,), (1,)), ((), ())),
                                  preferred_element_type=jnp.float32))

    h1 = two_dot(w1_ref) + b1_ref[0]
    h3 = two_dot(w3_ref) + b3_ref[0]
    p = h1 * h3
    h = p / (1.0 + jnp.exp(-p))
    out = lax.dot_general(h, w2_ref[0], (((1,), (1,)), ((), ())),
                          preferred_element_type=jnp.float32) + b2_ref[0]
    ys_ref[...] = out * w_ref[...]


def _ffn_body_alias(texp_ref, xs_ref, w_ref, w1_ref, b1_ref, w2_ref,
                    b2_ref, w3_ref, b3_ref, ysin_ref, ys_ref):
    _ffn_body(texp_ref, xs_ref, w_ref, w1_ref, b1_ref, w2_ref, b2_ref,
              w3_ref, b3_ref, ys_ref)


NTH = NT // 2


def _grouped_ffn_half1(texp, xs1, wrow, W1, b1, W2, b2, W3, b3):
    grid_spec = pltpu.PrefetchScalarGridSpec(
        num_scalar_prefetch=1,
        grid=(NTH,),
        in_specs=[
            pl.BlockSpec((TM, DP), lambda j, te: (j, 0)),
            pl.BlockSpec((TM, 1), lambda j, te: (j, 0)),
            pl.BlockSpec((1, F, D), lambda j, te: (te[j], 0, 0)),
            pl.BlockSpec((1, 1, F), lambda j, te: (te[j], 0, 0)),
            pl.BlockSpec((1, D, F), lambda j, te: (te[j], 0, 0)),
            pl.BlockSpec((1, 1, D), lambda j, te: (te[j], 0, 0)),
            pl.BlockSpec((1, F, D), lambda j, te: (te[j], 0, 0)),
            pl.BlockSpec((1, 1, F), lambda j, te: (te[j], 0, 0)),
        ],
        out_specs=pl.BlockSpec((TM, D), lambda j, te: (j, 0)),
    )
    return pl.pallas_call(
        _ffn_body,
        grid_spec=grid_spec,
        out_shape=jax.ShapeDtypeStruct((NP, D), jnp.float32),
    )(texp, xs1, wrow, W1, b1, W2, b2, W3, b3)


def _grouped_ffn_half2(texp, xs2, wrow, W1, b1, W2, b2, W3, b3, ys1):
    grid_spec = pltpu.PrefetchScalarGridSpec(
        num_scalar_prefetch=1,
        grid=(NTH,),
        in_specs=[
            pl.BlockSpec((TM, DP), lambda j, te: (j, 0)),
            pl.BlockSpec((TM, 1), lambda j, te: (j + NTH, 0)),
            pl.BlockSpec((1, F, D), lambda j, te: (te[j + NTH], 0, 0)),
            pl.BlockSpec((1, 1, F), lambda j, te: (te[j + NTH], 0, 0)),
            pl.BlockSpec((1, D, F), lambda j, te: (te[j + NTH], 0, 0)),
            pl.BlockSpec((1, 1, D), lambda j, te: (te[j + NTH], 0, 0)),
            pl.BlockSpec((1, F, D), lambda j, te: (te[j + NTH], 0, 0)),
            pl.BlockSpec((1, 1, F), lambda j, te: (te[j + NTH], 0, 0)),
            pl.BlockSpec((TM, D), lambda j, te: (j + NTH, 0)),
        ],
        out_specs=pl.BlockSpec((TM, D), lambda j, te: (j + NTH, 0)),
    )
    return pl.pallas_call(
        _ffn_body_alias,
        grid_spec=grid_spec,
        out_shape=jax.ShapeDtypeStruct((NP, D), jnp.float32),
        input_output_aliases={9: 0},
    )(texp, xs2, wrow, W1, b1, W2, b2, W3, b3, ys1)


def _shared_body(x_ref, w1_ref, b1_ref, w2_ref, b2_ref, w3_ref, b3_ref,
                 s_ref):
    xt = x_ref[...]
    h1 = lax.dot_general(xt, w1_ref[...], (((1,), (1,)), ((), ())),
                         preferred_element_type=jnp.float32) + b1_ref[...]
    h3 = lax.dot_general(xt, w3_ref[...], (((1,), (1,)), ((), ())),
                         preferred_element_type=jnp.float32) + b3_ref[...]
    p = h1 * h3
    h = p / (1.0 + jnp.exp(-p))
    s_ref[...] = lax.dot_general(h, w2_ref[...], (((1,), (1,)), ((), ())),
                                 preferred_element_type=jnp.float32) \
        + b2_ref[...]


_ET = 128


def _shared_ffn(xf, Ws1, bs1, Ws2, bs2, Ws3, bs3):
    return pl.pallas_call(
        _shared_body,
        grid=(T // _ET,),
        in_specs=[
            pl.BlockSpec((_ET, D), lambda i: (i, 0)),
            pl.BlockSpec((FS, D), lambda i: (0, 0)),
            pl.BlockSpec((FS,), lambda i: (0,)),
            pl.BlockSpec((D, FS), lambda i: (0, 0)),
            pl.BlockSpec((D,), lambda i: (0,)),
            pl.BlockSpec((FS, D), lambda i: (0, 0)),
            pl.BlockSpec((FS,), lambda i: (0,)),
        ],
        out_specs=pl.BlockSpec((_ET, D), lambda i: (i, 0)),
        out_shape=jax.ShapeDtypeStruct((T, D), jnp.float32),
    )(xf, Ws1, bs1, Ws2, bs2, Ws3, bs3)


_CTOK = 8
_TPW = T // NW


_FNC = _TPW // _CTOK


def _combine_body(pos_hbm, ys_hbm, s_hbm, y_hbm,
                  idxv, g0, g1, s0, s1, gs0, gs1, ss0, ss1):
    wid = lax.axis_index("s") * NC + lax.axis_index("c")
    tb0 = wid * _TPW
    pltpu.sync_copy(pos_hbm.at[pl.ds(tb0 * K, _TPW * K)], idxv)
    gb, sb = (g0, g1), (s0, s1)
    gsem, ssem = (gs0, gs1), (ss0, ss1)
    gcop, scop = [], []

    def combine_chunk(c):
        gcop[c].wait()
        scop[c].wait()
        gv, sv = gb[c % 2], sb[c % 2]
        for r in range(_CTOK):
            def body(c2, _, r=r, gv=gv, sv=sv):
                off = c2 * 16
                yv = (gv[2 * r, pl.ds(off, 16)]
                      + gv[2 * r + 1, pl.ds(off, 16)]
                      + sv[r, pl.ds(off, 16)])
                sv[r, pl.ds(off, 16)] = yv
                return 0
            lax.fori_loop(0, D // 16, body, 0)
        pltpu.sync_copy(sv, y_hbm.at[pl.ds(tb0 + c * _CTOK, _CTOK)])

    for ch in range(_FNC):
        gcop.append(pltpu.async_copy(
            ys_hbm.at[idxv.at[pl.ds(ch * _CTOK * K, _CTOK * K)]],
            gb[ch % 2], gsem[ch % 2]))
        scop.append(pltpu.async_copy(
            s_hbm.at[pl.ds(tb0 + ch * _CTOK, _CTOK)],
            sb[ch % 2], ssem[ch % 2]))
        if ch > 0:
            combine_chunk(ch - 1)
    combine_chunk(_FNC - 1)


def _combine(posf, ys, s):
    mesh = plsc.VectorSubcoreMesh(core_axis_name="c", subcore_axis_name="s")
    return pl.kernel(
        _combine_body,
        out_type=jax.ShapeDtypeStruct((T, D), jnp.float32),
        mesh=mesh,
        compiler_params=pltpu.CompilerParams(needs_layout_passes=False),
        scratch_types=[
            pltpu.VMEM((_TPW * K,), jnp.int32),
            pltpu.VMEM((_CTOK * K, D), jnp.float32),
            pltpu.VMEM((_CTOK * K, D), jnp.float32),
            pltpu.VMEM((_CTOK, D), jnp.float32),
            pltpu.VMEM((_CTOK, D), jnp.float32),
            pltpu.SemaphoreType.DMA,
            pltpu.SemaphoreType.DMA,
            pltpu.SemaphoreType.DMA,
            pltpu.SemaphoreType.DMA,
        ],
    )(posf, ys, s)


def kernel(x, gate_w, W1, b1, W2, b2, W3, b3, Ws1, bs1, Ws2, bs2, Ws3, bs3):
    shape = x.shape
    xf = x.reshape(-1, shape[-1])

    pos, wpair, texp2, xpack = _route(xf, gate_w)
    texp = texp2.reshape(NT)
    posf = pos.reshape(T * K)
    wf = wpair.reshape(T * K)

    srcv, wrow = _scatter(posf, wf)
    wrow2 = wrow.reshape(NP, 1)
    b1r, b2r, b3r = b1.reshape(E, 1, F), b2.reshape(E, 1, D), b3.reshape(E, 1, F)
    xs1 = _gather(srcv, xpack, 0)
    xs2 = _gather(srcv, xpack, NPH)
    ys1 = _grouped_ffn_half1(texp, xs1, wrow2, W1, b1r, W2, b2r, W3, b3r)
    ys = _grouped_ffn_half2(texp, xs2, wrow2, W1, b1r, W2, b2r, W3, b3r, ys1)
    s = _shared_ffn(xf, Ws1, bs1, Ws2, bs2, Ws3, bs3)
    y = _combine(posf, ys, s)
    return y.reshape(shape)
